# baseline (speedup 1.0000x reference)
import functools

import jax
import jax.numpy as jnp
from jax import lax
from jax.experimental import pallas as pl
from jax.experimental.pallas import tpu as pltpu
from jax.experimental.pallas import tpu_sc as plsc

_GCHUNK = 128


def _sc_gather(table, ids3):
    num_workers, chunks_per_w, chunk = ids3.shape
    depth = table.shape[1]
    out_rows = num_workers * chunks_per_w * chunk

    info = plsc.get_sparse_core_info()
    num_cores = info.num_cores

    mesh = plsc.VectorSubcoreMesh(core_axis_name="c", subcore_axis_name="s")

    @functools.partial(
        pl.kernel,
        mesh=mesh,
        out_type=jax.ShapeDtypeStruct((out_rows, 2 * depth), jnp.float32),
        scratch_types=[
            pltpu.VMEM((chunks_per_w, chunk), jnp.int32),
            pltpu.VMEM((2, chunk, depth), jnp.float32),
            pltpu.SemaphoreType.DMA,
            pltpu.SemaphoreType.DMA,
            pltpu.SemaphoreType.DMA,
        ],
        compiler_params=pltpu.CompilerParams(use_tc_tiling_on_sc=False),
    )
    def gather_kernel(table_hbm, idx_hbm, out_hbm, idx_v, rows_v,
                      gsem0, gsem1, ssem):
        wid = lax.axis_index("s") * num_cores + lax.axis_index("c")
        first_chunk = wid * chunks_per_w
        pltpu.sync_copy(idx_hbm.at[wid], idx_v)

        def out_rows_ref(j):
            off = pl.multiple_of((first_chunk + j) * chunk, chunk)
            return out_hbm.at[pl.ds(off, chunk), pl.ds(0, depth)]

        def wait_scatter(slot, j):
            pltpu.make_async_copy(rows_v.at[slot], out_rows_ref(j), ssem).wait()

        pltpu.async_copy(table_hbm.at[idx_v.at[0]], rows_v.at[0], gsem0)

        def body2(j, carry):
            slot = lax.rem(j, 2)
            nslot = 1 - slot

            @pl.when(j + 1 < chunks_per_w)
            def _():
                @pl.when(j >= 1)
                def _():
                    wait_scatter(nslot, j - 1)

                @pl.when(nslot == 0)
                def _():
                    pltpu.async_copy(
                        table_hbm.at[idx_v.at[j + 1]], rows_v.at[0], gsem0)

                @pl.when(nslot == 1)
                def _():
                    pltpu.async_copy(
                        table_hbm.at[idx_v.at[j + 1]], rows_v.at[1], gsem1)

            @pl.when(slot == 0)
            def _():
                pltpu.make_async_copy(
                    table_hbm.at[idx_v.at[0]], rows_v.at[0], gsem0).wait()

            @pl.when(slot == 1)
            def _():
                pltpu.make_async_copy(
                    table_hbm.at[idx_v.at[0]], rows_v.at[1], gsem1).wait()

            pltpu.async_copy(rows_v.at[slot], out_rows_ref(j), ssem)
            return carry

        lax.fori_loop(0, chunks_per_w, body2, 0)
        wait_scatter(lax.rem(chunks_per_w - 1, 2), chunks_per_w - 1)
        wait_scatter(lax.rem(chunks_per_w, 2), chunks_per_w - 2)

    return gather_kernel(table, ids3)


def _tc_project_norm(gathered, W2, rms_weight, d_model, block_rows,
                     total_out_rows, block_offset, prev=None):
    rows, width = gathered.shape
    grid = rows // block_rows

    def body(*refs):
        g_ref, w_ref, rw_ref, o_ref = refs[0], refs[1], refs[2], refs[-1]
        lane = lax.broadcasted_iota(jnp.int32, (block_rows, width), 1)
        g = jnp.where(lane < width // 2, g_ref[...], 0.0)
        y = lax.dot_general(
            g, w_ref[...], (((1,), (0,)), ((), ())),
            preferred_element_type=jnp.float32,
        )
        ms = jnp.mean(y * y, axis=-1, keepdims=True)
        o_ref[...] = y * lax.rsqrt(ms + 1e-8) * rw_ref[...]

    in_specs = [
        pl.BlockSpec((block_rows, width), lambda i: (i, 0)),
        pl.BlockSpec((width, d_model), lambda i: (0, 0)),
        pl.BlockSpec((1, d_model), lambda i: (0, 0)),
    ]
    args = [gathered, W2, rms_weight]
    aliases = {}
    if prev is not None:
        in_specs.append(pl.BlockSpec(memory_space=pl.ANY))
        args.append(prev)
        aliases = {3: 0}

    return pl.pallas_call(
        body,
        grid=(grid,),
        in_specs=in_specs,
        out_specs=pl.BlockSpec(
            (block_rows, d_model), lambda i: (i + block_offset, 0)),
        out_shape=jax.ShapeDtypeStruct((total_out_rows, d_model), jnp.float32),
        input_output_aliases=aliases,
        compiler_params=pltpu.CompilerParams(
            dimension_semantics=("arbitrary",),
        ),
    )(*args)


def kernel(token_ids, basin_coords, W, rms_weight):
    batch, seq = token_ids.shape
    d_model, depth = W.shape
    info = plsc.get_sparse_core_info()
    num_workers = info.num_cores * info.num_subcores

    ids = token_ids.reshape(-1).astype(jnp.int32)
    total = ids.shape[0]
    half = total // 2
    block_rows = 4096

    W2 = jnp.concatenate([W.T, jnp.zeros_like(W.T)], axis=0)
    rw2 = rms_weight.reshape(1, d_model)

    unit = num_workers * _GCHUNK
    total_chunks = total // unit
    split_sizes = [8, 14, 14, 14] if total_chunks == 50 else [total_chunks]

    out = None
    tok_off = 0
    for cpw_s in split_sizes:
        n_tok = cpw_s * unit
        ids_s = ids[tok_off:tok_off + n_tok].reshape(num_workers, -1, _GCHUNK)
        gathered_s = _sc_gather(basin_coords, ids_s)
        out = _tc_project_norm(gathered_s, W2, rw2, d_model, block_rows,
                               total, tok_off // block_rows, prev=out)
        tok_off += n_tok
    return out.reshape(batch, seq, d_model)


# speedup vs baseline: 1.0072x; 1.0072x over previous
import functools

import jax
import jax.numpy as jnp
from jax import lax
from jax.experimental import pallas as pl
from jax.experimental.pallas import tpu as pltpu
from jax.experimental.pallas import tpu_sc as plsc

_GCHUNK = 128


def _sc_gather(table, ids3):
    num_workers, chunks_per_w, chunk = ids3.shape
    depth = table.shape[1]
    out_rows = num_workers * chunks_per_w * chunk

    info = plsc.get_sparse_core_info()
    num_cores = info.num_cores

    mesh = plsc.VectorSubcoreMesh(core_axis_name="c", subcore_axis_name="s")

    @functools.partial(
        pl.kernel,
        mesh=mesh,
        out_type=jax.ShapeDtypeStruct((out_rows, 2 * depth), jnp.float32),
        scratch_types=[
            pltpu.VMEM((chunks_per_w, chunk), jnp.int32),
            pltpu.VMEM((2, chunk, depth), jnp.float32),
            pltpu.SemaphoreType.DMA,
            pltpu.SemaphoreType.DMA,
            pltpu.SemaphoreType.DMA,
        ],
        compiler_params=pltpu.CompilerParams(use_tc_tiling_on_sc=False),
    )
    def gather_kernel(table_hbm, idx_hbm, out_hbm, idx_v, rows_v,
                      gsem0, gsem1, ssem):
        wid = lax.axis_index("s") * num_cores + lax.axis_index("c")
        first_chunk = wid * chunks_per_w
        pltpu.sync_copy(idx_hbm.at[wid], idx_v)

        def out_rows_ref(j):
            off = pl.multiple_of((first_chunk + j) * chunk, chunk)
            return out_hbm.at[pl.ds(off, chunk), pl.ds(0, depth)]

        def wait_scatter(slot, j):
            pltpu.make_async_copy(rows_v.at[slot], out_rows_ref(j), ssem).wait()

        pltpu.async_copy(table_hbm.at[idx_v.at[0]], rows_v.at[0], gsem0)

        def body2(j, carry):
            slot = lax.rem(j, 2)
            nslot = 1 - slot

            @pl.when(j + 1 < chunks_per_w)
            def _():
                @pl.when(j >= 1)
                def _():
                    wait_scatter(nslot, j - 1)

                @pl.when(nslot == 0)
                def _():
                    pltpu.async_copy(
                        table_hbm.at[idx_v.at[j + 1]], rows_v.at[0], gsem0)

                @pl.when(nslot == 1)
                def _():
                    pltpu.async_copy(
                        table_hbm.at[idx_v.at[j + 1]], rows_v.at[1], gsem1)

            @pl.when(slot == 0)
            def _():
                pltpu.make_async_copy(
                    table_hbm.at[idx_v.at[0]], rows_v.at[0], gsem0).wait()

            @pl.when(slot == 1)
            def _():
                pltpu.make_async_copy(
                    table_hbm.at[idx_v.at[0]], rows_v.at[1], gsem1).wait()

            pltpu.async_copy(rows_v.at[slot], out_rows_ref(j), ssem)
            return carry

        lax.fori_loop(0, chunks_per_w, body2, 0)
        wait_scatter(lax.rem(chunks_per_w - 1, 2), chunks_per_w - 1)
        wait_scatter(lax.rem(chunks_per_w, 2), chunks_per_w - 2)

    return gather_kernel(table, ids3)


def _tc_project_norm(gathered, W2, rms_weight, d_model, block_rows,
                     total_out_rows, block_offset, prev=None):
    rows, width = gathered.shape
    grid = rows // block_rows

    def body(*refs):
        g_ref, w_ref, rw_ref, o_ref = refs[0], refs[1], refs[2], refs[-1]
        lane = lax.broadcasted_iota(jnp.int32, (block_rows, width), 1)
        g = jnp.where(lane < width // 2, g_ref[...], 0.0)
        y = lax.dot_general(
            g, w_ref[...], (((1,), (0,)), ((), ())),
            preferred_element_type=jnp.float32,
        )
        ms = jnp.mean(y * y, axis=-1, keepdims=True)
        o_ref[...] = y * lax.rsqrt(ms + 1e-8) * rw_ref[...]

    in_specs = [
        pl.BlockSpec((block_rows, width), lambda i: (i, 0)),
        pl.BlockSpec((width, d_model), lambda i: (0, 0)),
        pl.BlockSpec((1, d_model), lambda i: (0, 0)),
    ]
    args = [gathered, W2, rms_weight]
    aliases = {}
    if prev is not None:
        in_specs.append(pl.BlockSpec(memory_space=pl.ANY))
        args.append(prev)
        aliases = {3: 0}

    return pl.pallas_call(
        body,
        grid=(grid,),
        in_specs=in_specs,
        out_specs=pl.BlockSpec(
            (block_rows, d_model), lambda i: (i + block_offset, 0)),
        out_shape=jax.ShapeDtypeStruct((total_out_rows, d_model), jnp.float32),
        input_output_aliases=aliases,
        compiler_params=pltpu.CompilerParams(
            dimension_semantics=("arbitrary",),
        ),
    )(*args)


def kernel(token_ids, basin_coords, W, rms_weight):
    batch, seq = token_ids.shape
    d_model, depth = W.shape
    info = plsc.get_sparse_core_info()
    num_workers = info.num_cores * info.num_subcores

    ids = token_ids.reshape(-1).astype(jnp.int32)
    total = ids.shape[0]
    half = total // 2
    block_rows = 4096

    W2 = jnp.concatenate([W.T, jnp.zeros_like(W.T)], axis=0)
    rw2 = rms_weight.reshape(1, d_model)

    unit = num_workers * _GCHUNK
    total_chunks = total // unit
    split_sizes = [12, 38] if total_chunks == 50 else [total_chunks]

    out = None
    tok_off = 0
    for cpw_s in split_sizes:
        n_tok = cpw_s * unit
        ids_s = ids[tok_off:tok_off + n_tok].reshape(num_workers, -1, _GCHUNK)
        gathered_s = _sc_gather(basin_coords, ids_s)
        out = _tc_project_norm(gathered_s, W2, rw2, d_model, block_rows,
                               total, tok_off // block_rows, prev=out)
        tok_off += n_tok
    return out.reshape(batch, seq, d_model)


# speedup vs baseline: 1.0089x; 1.0017x over previous
import functools

import jax
import jax.numpy as jnp
from jax import lax
from jax.experimental import pallas as pl
from jax.experimental.pallas import tpu as pltpu
from jax.experimental.pallas import tpu_sc as plsc

_GCHUNK = 128


def _sc_gather(table, ids3):
    num_workers, chunks_per_w, chunk = ids3.shape
    depth = table.shape[1]
    out_rows = num_workers * chunks_per_w * chunk

    info = plsc.get_sparse_core_info()
    num_cores = info.num_cores

    mesh = plsc.VectorSubcoreMesh(core_axis_name="c", subcore_axis_name="s")

    @functools.partial(
        pl.kernel,
        mesh=mesh,
        out_type=jax.ShapeDtypeStruct((out_rows, 2 * depth), jnp.float32),
        scratch_types=[
            pltpu.VMEM((chunks_per_w, chunk), jnp.int32),
            pltpu.VMEM((2, chunk, depth), jnp.float32),
            pltpu.SemaphoreType.DMA,
            pltpu.SemaphoreType.DMA,
            pltpu.SemaphoreType.DMA,
        ],
        compiler_params=pltpu.CompilerParams(use_tc_tiling_on_sc=False),
    )
    def gather_kernel(table_hbm, idx_hbm, out_hbm, idx_v, rows_v,
                      gsem0, gsem1, ssem):
        wid = lax.axis_index("s") * num_cores + lax.axis_index("c")
        first_chunk = wid * chunks_per_w
        pltpu.sync_copy(idx_hbm.at[wid], idx_v)

        def out_rows_ref(j):
            off = pl.multiple_of((first_chunk + j) * chunk, chunk)
            return out_hbm.at[pl.ds(off, chunk), pl.ds(0, depth)]

        def wait_scatter(slot, j):
            pltpu.make_async_copy(rows_v.at[slot], out_rows_ref(j), ssem).wait()

        pltpu.async_copy(table_hbm.at[idx_v.at[0]], rows_v.at[0], gsem0)

        def body2(j, carry):
            slot = lax.rem(j, 2)
            nslot = 1 - slot

            @pl.when(j + 1 < chunks_per_w)
            def _():
                @pl.when(j >= 1)
                def _():
                    wait_scatter(nslot, j - 1)

                @pl.when(nslot == 0)
                def _():
                    pltpu.async_copy(
                        table_hbm.at[idx_v.at[j + 1]], rows_v.at[0], gsem0)

                @pl.when(nslot == 1)
                def _():
                    pltpu.async_copy(
                        table_hbm.at[idx_v.at[j + 1]], rows_v.at[1], gsem1)

            @pl.when(slot == 0)
            def _():
                pltpu.make_async_copy(
                    table_hbm.at[idx_v.at[0]], rows_v.at[0], gsem0).wait()

            @pl.when(slot == 1)
            def _():
                pltpu.make_async_copy(
                    table_hbm.at[idx_v.at[0]], rows_v.at[1], gsem1).wait()

            pltpu.async_copy(rows_v.at[slot], out_rows_ref(j), ssem)
            return carry

        lax.fori_loop(0, chunks_per_w, body2, 0)
        wait_scatter(lax.rem(chunks_per_w - 1, 2), chunks_per_w - 1)
        if chunks_per_w >= 2:
            wait_scatter(lax.rem(chunks_per_w, 2), chunks_per_w - 2)

    return gather_kernel(table, ids3)


def _tc_project_norm(gathered, W2, rms_weight, d_model, block_rows,
                     total_out_rows, block_offset, prev=None):
    rows, width = gathered.shape
    grid = rows // block_rows

    def body(*refs):
        g_ref, w_ref, rw_ref, o_ref = refs[0], refs[1], refs[2], refs[-1]
        lane = lax.broadcasted_iota(jnp.int32, (block_rows, width), 1)
        g = jnp.where(lane < width // 2, g_ref[...], 0.0)
        y = lax.dot_general(
            g, w_ref[...], (((1,), (0,)), ((), ())),
            preferred_element_type=jnp.float32,
        )
        ms = jnp.mean(y * y, axis=-1, keepdims=True)
        o_ref[...] = y * lax.rsqrt(ms + 1e-8) * rw_ref[...]

    in_specs = [
        pl.BlockSpec((block_rows, width), lambda i: (i, 0)),
        pl.BlockSpec((width, d_model), lambda i: (0, 0)),
        pl.BlockSpec((1, d_model), lambda i: (0, 0)),
    ]
    args = [gathered, W2, rms_weight]
    aliases = {}
    if prev is not None:
        in_specs.append(pl.BlockSpec(memory_space=pl.ANY))
        args.append(prev)
        aliases = {3: 0}

    return pl.pallas_call(
        body,
        grid=(grid,),
        in_specs=in_specs,
        out_specs=pl.BlockSpec(
            (block_rows, d_model), lambda i: (i + block_offset, 0)),
        out_shape=jax.ShapeDtypeStruct((total_out_rows, d_model), jnp.float32),
        input_output_aliases=aliases,
        compiler_params=pltpu.CompilerParams(
            dimension_semantics=("arbitrary",),
        ),
    )(*args)


def kernel(token_ids, basin_coords, W, rms_weight):
    batch, seq = token_ids.shape
    d_model, depth = W.shape
    info = plsc.get_sparse_core_info()
    num_workers = info.num_cores * info.num_subcores

    ids = token_ids.reshape(-1).astype(jnp.int32)
    total = ids.shape[0]
    block_rows = 4096

    W2 = jnp.concatenate([W.T, jnp.zeros_like(W.T)], axis=0)
    rw2 = rms_weight.reshape(1, d_model)

    unit = num_workers * _GCHUNK
    total_chunks = total // unit
    split_sizes = [12, 38] if total_chunks == 50 else [total_chunks]

    out = None
    tok_off = 0
    for cpw_s in split_sizes:
        n_tok = cpw_s * unit
        ids_s = ids[tok_off:tok_off + n_tok].reshape(num_workers, -1, _GCHUNK)
        gathered_s = _sc_gather(basin_coords, ids_s)
        out = _tc_project_norm(gathered_s, W2, rw2, d_model, block_rows,
                               total, tok_off // block_rows, prev=out)
        tok_off += n_tok
    return out.reshape(batch, seq, d_model)
